# Initial kernel scaffold; baseline (speedup 1.0000x reference)
import jax
import jax.numpy as jnp
from jax.experimental import pallas as pl


def kernel(p, x, params):
    raise NotImplementedError("write your pallas kernel here")


# speedup vs baseline: 7.6998x; 7.6998x over previous
import functools

import jax
import jax.numpy as jnp
from jax import lax
from jax.experimental import pallas as pl
from jax.experimental.pallas import tpu as pltpu
from jax.experimental.pallas import tpu_sc as plsc

_K = 32
_MM_R = 512

def _dot(a, b):
    return jnp.dot(a, b, preferred_element_type=jnp.float32)


def _sc_gather(table, idx):
    V, D = table.shape
    Bn = idx.shape[0]
    info = plsc.get_sparse_core_info()
    NC, NS = info.num_cores, info.num_subcores
    NW = NC * NS
    assert Bn % (8 * NW) == 0 and D % 128 == 0
    b_per_w = Bn // NW
    nch = 1
    while (b_per_w // nch) * D > 96 * 1024:
        nch *= 2
    rpc = b_per_w // nch
    mesh = plsc.VectorSubcoreMesh(core_axis_name="c", subcore_axis_name="s")

    @functools.partial(
        pl.kernel, mesh=mesh,
        out_type=jax.ShapeDtypeStruct((Bn, D), jnp.float32),
        scratch_types=[
            pltpu.VMEM((b_per_w,), jnp.int32),
            pltpu.VMEM((rpc, D), jnp.float32),
            pltpu.SemaphoreType.DMA,
        ],
    )
    def k(table_hbm, idx_hbm, out_hbm, idx_v, rows_v, sem):
        wid = lax.axis_index("s") * NC + lax.axis_index("c")
        base = wid * b_per_w
        pltpu.sync_copy(idx_hbm.at[pl.ds(base, b_per_w)], idx_v)
        for c in range(nch):
            iv = idx_v if nch == 1 else idx_v.at[pl.ds(c * rpc, rpc)]
            pltpu.async_copy(table_hbm.at[iv], rows_v, sem).wait()
            pltpu.sync_copy(rows_v, out_hbm.at[pl.ds(base + c * rpc, rpc)])

    return k(table, idx)


def _fps_kernel(pT_ref, q_ref, *, M):
    Bb, _, N = pT_ref.shape
    px = pT_ref[:, 0, :]
    py = pT_ref[:, 1, :]
    pz = pT_ref[:, 2, :]
    iota = lax.broadcasted_iota(jnp.int32, (Bb, N), 1)

    def body(i, carry):
        dists, far = carry
        sel = iota == far
        cx = jnp.sum(jnp.where(sel, px, 0.0), axis=1, keepdims=True)
        cy = jnp.sum(jnp.where(sel, py, 0.0), axis=1, keepdims=True)
        cz = jnp.sum(jnp.where(sel, pz, 0.0), axis=1, keepdims=True)
        q_ref[:, pl.ds(i, 1), :] = jnp.concatenate(
            [cx, cy, cz], axis=1)[:, None, :]
        d = (px - cx) ** 2 + (py - cy) ** 2 + (pz - cz) ** 2
        dists = jnp.minimum(dists, d)
        m = jnp.max(dists, axis=1, keepdims=True)
        far = jnp.min(jnp.where(dists == m, iota, N), axis=1, keepdims=True)
        return dists, far

    d0 = jnp.full((Bb, N), 1e10, jnp.float32)
    f0 = jnp.zeros((Bb, 1), jnp.int32)
    lax.fori_loop(0, M, body, (d0, f0))


def _fps(pT, M):
    B = pT.shape[0]
    return pl.pallas_call(
        functools.partial(_fps_kernel, M=M),
        out_shape=jax.ShapeDtypeStruct((B, M, 3), jnp.float32),
    )(pT)


def _ballq_kernel(q_ref, pT_ref, idx_ref, *, r2, N):
    b = pl.program_id(0)
    q = q_ref[0]
    qx, qy, qz = q[:, 0:1], q[:, 1:2], q[:, 2:3]
    px = pT_ref[0, 0:1, :]
    py = pT_ref[0, 1:2, :]
    pz = pT_ref[0, 2:3, :]
    d = (qx - px) ** 2 + (qy - py) ** 2 + (qz - pz) ** 2
    iota = lax.broadcasted_iota(jnp.int32, d.shape, 1)
    cand = jnp.where(d <= r2, iota, N)
    cols = []
    for _ in range(_K):
        m = jnp.min(cand, axis=1, keepdims=True)
        cols.append(m)
        cand = jnp.where(cand == m, N, cand)
    out = jnp.concatenate(cols, axis=1)
    first = out[:, 0:1]
    out = jnp.where(out == N, first, out)
    idx_ref[0] = out + b * N


def _ballq(q, pT, radius):
    B, M, _ = q.shape
    N = pT.shape[2]
    R = min(128, M)
    return pl.pallas_call(
        functools.partial(_ballq_kernel, r2=radius * radius, N=N),
        grid=(B, M // R),
        in_specs=[
            pl.BlockSpec((1, R, 3), lambda b, t: (b, t, 0)),
            pl.BlockSpec((1, 3, N), lambda b, t: (b, 0, 0)),
        ],
        out_specs=pl.BlockSpec((1, R, _K), lambda b, t: (b, t, 0)),
        out_shape=jax.ShapeDtypeStruct((B, M, _K), jnp.int32),
    )(q, pT)


def _fp3nn_kernel(q_ref, pT_ref, idx_ref, w_ref, *, Ns):
    b = pl.program_id(0)
    q = q_ref[0]
    qx, qy, qz = q[:, 0:1], q[:, 1:2], q[:, 2:3]
    px = pT_ref[0, 0:1, :]
    py = pT_ref[0, 1:2, :]
    pz = pT_ref[0, 2:3, :]
    d = (qx - px) ** 2 + (qy - py) ** 2 + (qz - pz) ** 2
    iota = lax.broadcasted_iota(jnp.int32, d.shape, 1)
    idxs, dists = [], []
    for _ in range(3):
        m = jnp.min(d, axis=1, keepdims=True)
        i = jnp.min(jnp.where(d == m, iota, Ns), axis=1, keepdims=True)
        idxs.append(i)
        dists.append(m)
        d = jnp.where(iota == i, 1e30, d)
    dist = jnp.maximum(jnp.concatenate(dists, axis=1), 1e-10)
    w = 1.0 / dist
    w = w / jnp.sum(w, axis=1, keepdims=True)
    idx_ref[0] = jnp.concatenate(idxs, axis=1) + b * Ns
    w_ref[0] = w


def _fp3nn(q_dst, pT_src):
    B, Nd, _ = q_dst.shape
    Ns = pT_src.shape[2]
    R = min(128, Nd)
    return pl.pallas_call(
        functools.partial(_fp3nn_kernel, Ns=Ns),
        grid=(B, Nd // R),
        in_specs=[
            pl.BlockSpec((1, R, 3), lambda b, t: (b, t, 0)),
            pl.BlockSpec((1, 3, Ns), lambda b, t: (b, 0, 0)),
        ],
        out_specs=[
            pl.BlockSpec((1, R, 3), lambda b, t: (b, t, 0)),
            pl.BlockSpec((1, R, 3), lambda b, t: (b, t, 0)),
        ],
        out_shape=[
            jax.ShapeDtypeStruct((B, Nd, 3), jnp.int32),
            jax.ShapeDtypeStruct((B, Nd, 3), jnp.float32),
        ],
    )(q_dst, pT_src)


def _stats(y):
    C = y.shape[-1]
    eye = jnp.eye(C, dtype=jnp.float32)
    yd = lax.dot_general(y, eye, (((y.ndim - 1,), (0,)), ((), ())),
                         precision=lax.Precision.HIGHEST,
                         preferred_element_type=jnp.float32)
    axes = tuple(range(y.ndim - 1))
    mu = jnp.mean(yd, axis=axes)
    sig = jnp.sqrt(jnp.var(yd, axis=axes) + 1e-5)
    return mu.reshape(1, -1), sig.reshape(1, -1)


def _lead_grid(lead, C):
    if len(lead) == 3:
        B_, M_, K_ = lead
        Rq = max(1, min(_MM_R // K_, M_))
        grid = (B_, M_ // Rq)
        R = Rq * K_
        imap = lambda b, t, _n=M_ // Rq: (b * _n + t, 0)
        out_spec = pl.BlockSpec((1, Rq, K_, C), lambda b, t: (b, t, 0, 0))
        out_shape = jax.ShapeDtypeStruct((B_, M_, K_, C), jnp.float32)
    else:
        B_, Nd_ = lead
        R = min(_MM_R, Nd_)
        grid = (B_, Nd_ // R)
        imap = lambda b, t, _n=Nd_ // R: (b * _n + t, 0)
        out_spec = pl.BlockSpec((1, R, C), lambda b, t: (b, t, 0))
        out_shape = jax.ShapeDtypeStruct((B_, Nd_, C), jnp.float32)
    return grid, R, imap, out_spec, out_shape


def _sal1_kernel(g_ref, qb_ref, wpad_ref, y_ref, *, Cx):
    g = g_ref[...]
    qb = qb_ref[...]
    R, Cpad = g.shape
    qpad = jnp.concatenate(
        [jnp.zeros((R, Cx), jnp.float32), qb,
         jnp.zeros((R, Cpad - Cx - 3), jnp.float32)], axis=1)
    y_ref[...] = _dot(g - qpad, wpad_ref[...]).reshape(y_ref.shape)


def _sal1(grows, qb, wpad, Cx, lead):
    Rows, Cpad = grows.shape
    C = wpad.shape[1]
    grid, R, imap, out_spec, out_shape = _lead_grid(lead, C)
    y = pl.pallas_call(
        functools.partial(_sal1_kernel, Cx=Cx),
        grid=grid,
        in_specs=[
            pl.BlockSpec((R, Cpad), imap),
            pl.BlockSpec((R, 3), imap),
            pl.BlockSpec((Cpad, C), lambda b, t: (0, 0)),
        ],
        out_specs=out_spec,
        out_shape=out_shape,
    )(grows, qb, wpad)
    mu, sig = _stats(y)
    return y, mu, sig


def _bnmm_kernel(y_ref, mu_ref, sig_ref, w_ref, yo_ref):
    z = jnp.maximum((y_ref[...] - mu_ref[...]) / sig_ref[...], 0.0)
    yo_ref[...] = _dot(z, w_ref[...]).reshape(yo_ref.shape)


def _bnmm(y, mu, sig, w, lead):
    y = y.reshape(-1, y.shape[-1])
    Rows, Cin = y.shape
    Cout = w.shape[1]
    grid, R, imap, out_spec, out_shape = _lead_grid(lead, Cout)
    yo = pl.pallas_call(
        _bnmm_kernel,
        grid=grid,
        in_specs=[
            pl.BlockSpec((R, Cin), imap),
            pl.BlockSpec((1, Cin), lambda b, t: (0, 0)),
            pl.BlockSpec((1, Cin), lambda b, t: (0, 0)),
            pl.BlockSpec((Cin, Cout), lambda b, t: (0, 0)),
        ],
        out_specs=out_spec,
        out_shape=out_shape,
    )(y, mu, sig, w)
    mu2, sig2 = _stats(yo)
    return yo, mu2, sig2


def _bnpool_kernel(y_ref, mu_ref, sig_ref, x_ref):
    z = jnp.maximum((y_ref[...] - mu_ref[...]) / sig_ref[...], 0.0)
    x_ref[...] = jnp.max(z, axis=2)


def _bnpool(y4d, mu, sig):
    B_, M_, K_, C = y4d.shape
    Rq = min(64, M_)
    return pl.pallas_call(
        _bnpool_kernel,
        grid=(B_, M_ // Rq),
        in_specs=[
            pl.BlockSpec((1, Rq, K_, C), lambda b, t: (b, t, 0, 0)),
            pl.BlockSpec((1, 1, 1, C), lambda b, t: (0, 0, 0, 0)),
            pl.BlockSpec((1, 1, 1, C), lambda b, t: (0, 0, 0, 0)),
        ],
        out_specs=pl.BlockSpec((1, Rq, C), lambda b, t: (b, t, 0)),
        out_shape=jax.ShapeDtypeStruct((B_, M_, C), jnp.float32),
    )(y4d, mu.reshape(1, 1, 1, C), sig.reshape(1, 1, 1, C))


def _bnfin_kernel(y_ref, mu_ref, sig_ref, x_ref):
    x_ref[...] = jnp.maximum((y_ref[...] - mu_ref[...]) / sig_ref[...], 0.0)


def _bnfin(y3d, mu, sig):
    B_, Nd_, C = y3d.shape
    R = min(_MM_R, Nd_)
    return pl.pallas_call(
        _bnfin_kernel,
        grid=(B_, Nd_ // R),
        in_specs=[
            pl.BlockSpec((1, R, C), lambda b, t: (b, t, 0)),
            pl.BlockSpec((1, 1, C), lambda b, t: (0, 0, 0)),
            pl.BlockSpec((1, 1, C), lambda b, t: (0, 0, 0)),
        ],
        out_specs=pl.BlockSpec((1, R, C), lambda b, t: (b, t, 0)),
        out_shape=jax.ShapeDtypeStruct((B_, Nd_, C), jnp.float32),
    )(y3d, mu.reshape(1, 1, C), sig.reshape(1, 1, C))


def _fpl1_kernel(g_ref, w3_ref, skip_ref, wa_ref, wb_ref, y_ref):
    g = g_ref[...]
    w3 = w3_ref[...]
    interp = jnp.sum(g * w3[:, :, None], axis=1)
    y = _dot(interp, wa_ref[...])
    if skip_ref is not None:
        y = y + _dot(skip_ref[...], wb_ref[...])
    y_ref[...] = y.reshape(y_ref.shape)


def _fpl1(g3d, w3, skip, wa, wb, lead):
    Rows, _, C1 = g3d.shape
    Cout = wa.shape[1]
    grid, R, imap, out_spec, out_shape = _lead_grid(lead, Cout)
    imap3 = lambda b, t: imap(b, t) + (0,)
    in_specs = [
        pl.BlockSpec((R, 3, C1), imap3),
        pl.BlockSpec((R, 3), imap),
    ]
    args = [g3d, w3]
    if skip is not None:
        C2 = skip.shape[1]
        in_specs += [
            pl.BlockSpec((R, C2), imap),
            pl.BlockSpec((C1, Cout), lambda b, t: (0, 0)),
            pl.BlockSpec((C2, Cout), lambda b, t: (0, 0)),
        ]
        args += [skip, wa, wb]
        body = _fpl1_kernel
    else:
        in_specs += [pl.BlockSpec((C1, Cout), lambda b, t: (0, 0))]
        args += [wa]

        def body(g_ref, w3_ref, wa_ref, y_ref):
            return _fpl1_kernel(g_ref, w3_ref, None, wa_ref, None, y_ref)

    y = pl.pallas_call(
        body,
        grid=grid,
        in_specs=in_specs,
        out_specs=out_spec,
        out_shape=out_shape,
    )(*args)
    mu, sig = _stats(y)
    return y, mu, sig


def _fc_kernel(y_ref, mu_ref, sig_ref, w_ref, b_ref, o_ref):
    z = jnp.maximum((y_ref[...] - mu_ref[...]) / sig_ref[...], 0.0)
    o_ref[...] = _dot(z, w_ref[...]) + b_ref[...]


def _fc(y, mu, sig, w, bvec):
    Rows, Cin = y.shape
    Cout = w.shape[1]
    R = min(_MM_R, Rows)
    G = Rows // R
    return pl.pallas_call(
        _fc_kernel,
        grid=(G,),
        in_specs=[
            pl.BlockSpec((R, Cin), lambda i: (i, 0)),
            pl.BlockSpec((1, Cin), lambda i: (0, 0)),
            pl.BlockSpec((1, Cin), lambda i: (0, 0)),
            pl.BlockSpec((Cin, Cout), lambda i: (0, 0)),
            pl.BlockSpec((1, Cout), lambda i: (0, 0)),
        ],
        out_specs=pl.BlockSpec((R, Cout), lambda i: (i, 0)),
        out_shape=jax.ShapeDtypeStruct((Rows, Cout), jnp.float32),
    )(y, mu, sig, w, bvec.reshape(1, Cout))


def _pad128(c):
    return (c + 127) // 128 * 128


def kernel(p, x, params):
    B, N, _ = p.shape
    cur_p = p
    cur_pT = jnp.transpose(p, (0, 2, 1))
    cur_x = jnp.transpose(x, (0, 2, 1))

    sa_specs = [
        ("sa1", 1024, 0.1),
        ("sa2", 256, 0.2),
        ("sa3", 64, 0.4),
        ("sa4", 16, 0.8),
    ]
    qs, xs = [], []
    for name, M, radius in sa_specs:
        Nl = cur_p.shape[1]
        Cx = cur_x.shape[2]
        Cpad = _pad128(Cx + 3)
        q = _fps(cur_pT, M)
        idx = _ballq(q, cur_pT, radius)
        G = jnp.concatenate(
            [cur_x, cur_p,
             jnp.zeros((B, Nl, Cpad - Cx - 3), jnp.float32)],
            axis=-1).reshape(B * Nl, Cpad)
        rows = _sc_gather(G, idx.reshape(-1))
        qb = jnp.broadcast_to(q[:, :, None, :],
                              (B, M, _K, 3)).reshape(-1, 3)
        W0 = params[f"{name}_W0"]
        C0 = W0.shape[1]
        wpad = jnp.concatenate(
            [W0, jnp.zeros((Cpad - W0.shape[0], C0), jnp.float32)], axis=0)
        lead = (B, M, _K)
        y, mu, sig = _sal1(rows, qb, wpad, Cx, lead)
        y, mu, sig = _bnmm(y, mu, sig, params[f"{name}_W1"], lead)
        y, mu, sig = _bnmm(y, mu, sig, params[f"{name}_W2"], lead)
        xl = _bnpool(y, mu, sig)
        qs.append(q)
        xs.append(xl)
        cur_p = q
        cur_pT = jnp.transpose(q, (0, 2, 1))
        cur_x = xl

    fp_specs = [
        ("fp1", qs[3], qs[2], xs[2]),
        ("fp2", qs[2], qs[1], xs[1]),
        ("fp3", qs[1], qs[0], xs[0]),
        ("fp4", qs[0], p, None),
    ]
    cur = xs[3]
    for name, p_src, p_dst, skip in fp_specs:
        Ns = p_src.shape[1]
        Nd = p_dst.shape[1]
        C1 = cur.shape[-1]
        idx3, w3 = _fp3nn(p_dst, jnp.transpose(p_src, (0, 2, 1)))
        g = _sc_gather(cur.reshape(-1, C1), idx3.reshape(-1))
        W0 = params[f"{name}_W0"]
        wa = W0[:C1]
        wb = W0[C1:] if skip is not None else None
        skip2d = skip.reshape(-1, skip.shape[-1]) if skip is not None else None
        lead = (B, Nd)
        y, mu, sig = _fpl1(g.reshape(B * Nd, 3, C1),
                           w3.reshape(-1, 3), skip2d, wa, wb, lead)
        y, mu, sig = _bnmm(y, mu, sig, params[f"{name}_W1"], lead)
        cur = _bnfin(y, mu, sig)

    zero = jnp.zeros((1, cur.shape[-1]), jnp.float32)
    one = jnp.ones((1, cur.shape[-1]), jnp.float32)
    y, mu, sig = _bnmm(cur, zero, one, params["mlp_W0"], (B, N))
    y, mu, sig = _bnmm(y, mu, sig, params["mlp_W1"], (B, N))
    out = _fc(y.reshape(B * N, -1), mu, sig, params["fc_W"], params["fc_b"])
    return jnp.transpose(out.reshape(B, N, -1), (0, 2, 1))
